# baseline (speedup 1.0000x reference)
import jax
import jax.numpy as jnp
from jax.experimental import pallas as pl
from jax.experimental.pallas import tpu as pltpu

S = 2048
D = 768
H = 12
HD = 64
K = 64
BLK = 512
WIN = 576
CHK = 128
CWIN = 192
NBLK = S // BLK
SQRT_HD = 8.0
EXP2_SCALE = 1.4426950408889634 / SQRT_HD


def _chunk_geometry():
    chunks = []
    for i in range(NBLK):
        r0 = BLK * i
        c0 = min(max(r0 - K // 2, 0), S - WIN)
        for ci in range(BLK // CHK):
            gr = r0 + CHK * ci
            cs = min(max(gr - K // 2 - c0, 0), WIN - CWIN)
            d0 = c0 + cs - gr
            if c0 + cs == 0 and gr == 0:
                v = 0
            elif c0 + cs == S - CWIN:
                v = 2
            else:
                assert d0 == -K // 2, (i, ci, d0)
                v = 1
            chunks.append((i, ci, r0, c0, gr, cs, v))
    return chunks


_CHUNKS = _chunk_geometry()


def _bias_chunk(r0, c0):
    rows = r0 + jax.lax.broadcasted_iota(jnp.int32, (CHK, CWIN), 0)
    cols = c0 + jax.lax.broadcasted_iota(jnp.int32, (CHK, CWIN), 1)
    off = cols - rows
    valid = (off >= -(K // 2)) & (off <= K // 2 - 1)
    rowsf = rows.astype(jnp.float32)
    cnt = jnp.where(cols == 0, jnp.maximum(33.0 - rowsf, 1.0), 1.0)
    cnt = jnp.where(cols == S - 1, jnp.maximum(rowsf - (S - 33.0), 1.0), cnt)
    return jnp.where(valid, SQRT_HD * jnp.log(cnt), -1e30)


def _fused_kernel(x_ref, win_ref, wout_ref, bout_ref, out_ref,
                  h_ref, f_ref, bias_ref):
    h_ref[...] = jnp.dot(x_ref[...].astype(jnp.bfloat16), win_ref[...],
                         preferred_element_type=jnp.float32
                         ).astype(jnp.bfloat16)

    bias_ref[0] = _bias_chunk(0, 0)
    bias_ref[1] = _bias_chunk(CHK, CHK - K // 2)
    bias_ref[2] = _bias_chunk(S - CHK, S - CWIN)

    for i in range(NBLK):
        r0 = BLK * i
        c0 = min(max(r0 - K // 2, 0), S - WIN)
        for hd in range(H):
            lo, hi = hd * HD, (hd + 1) * HD
            qh = h_ref[r0:r0 + BLK, lo:hi]
            kw = h_ref[c0:c0 + WIN, lo:hi]
            st = jax.lax.dot_general(
                qh, kw, (((1,), (1,)), ((), ())),
                preferred_element_type=jnp.float32)
            for (bi, ci, _r0, _c0, gr, cs, v) in _CHUNKS:
                if bi != i:
                    continue
                rl = CHK * ci
                sub = st[rl:rl + CHK, cs:cs + CWIN] + bias_ref[v]
                m = jnp.max(sub, axis=1, keepdims=True)
                p = jnp.exp2((sub - m) * EXP2_SCALE)
                z = jnp.sum(p, axis=1, keepdims=True)
                kc = h_ref[c0 + cs:c0 + cs + CWIN, lo:hi]
                fu = jnp.dot(p.astype(jnp.bfloat16), kc,
                             preferred_element_type=jnp.float32)
                f_ref[gr:gr + CHK, lo:hi] = (fu / z).astype(jnp.bfloat16)

    out_ref[...] = (jnp.dot(f_ref[...], wout_ref[...],
                            preferred_element_type=jnp.float32)
                    + bout_ref[...] + x_ref[...])


@jax.jit
def kernel(x, W_in, W_out, b_out):
    x2 = x.reshape(S, D)
    out = pl.pallas_call(
        _fused_kernel,
        out_shape=jax.ShapeDtypeStruct((S, D), jnp.float32),
        scratch_shapes=[
            pltpu.VMEM((S, D), jnp.bfloat16),
            pltpu.VMEM((S, D), jnp.bfloat16),
            pltpu.VMEM((3, CHK, CWIN), jnp.float32),
        ],
    )(x2, W_in.astype(jnp.bfloat16), W_out.astype(jnp.bfloat16),
      b_out.reshape(1, D))
    return out.reshape(1, S, D)


# speedup vs baseline: 1.2960x; 1.2960x over previous
import jax
import jax.numpy as jnp
from jax.experimental import pallas as pl
from jax.experimental.pallas import tpu as pltpu

S = 2048
D = 768
H = 12
HD = 64
K = 64
BLK = 512
WIN = 576
NBLK = S // BLK
SQRT_HD = 8.0
EXP2_SCALE = 1.4426950408889634 / SQRT_HD


def _bias_t(r0, c0):
    rows = r0 + jax.lax.broadcasted_iota(jnp.int32, (BLK, WIN), 0)
    cols = c0 + jax.lax.broadcasted_iota(jnp.int32, (BLK, WIN), 1)
    off = cols - rows
    valid = (off >= -(K // 2)) & (off <= K // 2 - 1)
    rowsf = rows.astype(jnp.float32)
    cnt = jnp.where(cols == 0, jnp.maximum(33.0 - rowsf, 1.0), 1.0)
    cnt = jnp.where(cols == S - 1, jnp.maximum(rowsf - (S - 33.0), 1.0), cnt)
    return jnp.where(valid, SQRT_HD * jnp.log(cnt), -1e30)


def _fused_kernel(x_ref, win_ref, wout_ref, bout_ref, out_ref,
                  h_ref, f_ref, bias_ref):
    h_ref[...] = jnp.dot(x_ref[...].astype(jnp.bfloat16), win_ref[...],
                         preferred_element_type=jnp.float32
                         ).astype(jnp.bfloat16)

    bias_ref[0] = _bias_t(0, 0)
    bias_ref[1] = _bias_t(BLK, BLK - K // 2)
    bias_ref[2] = _bias_t(S - BLK, S - WIN)

    def body(i, carry):
        r0 = i * BLK
        c0 = pl.multiple_of(jnp.clip(r0 - K // 2, 0, S - WIN), 32)
        widx = jnp.where(i == 0, 0, jnp.where(i == NBLK - 1, 2, 1))
        q = h_ref[pl.ds(r0, BLK), :]
        kw = h_ref[pl.ds(c0, WIN), :]
        bias = bias_ref[widx]

        for hd in range(H):
            qh = q[:, hd * HD:(hd + 1) * HD]
            kh = kw[:, hd * HD:(hd + 1) * HD]
            st = jax.lax.dot_general(
                qh, kh, (((1,), (1,)), ((), ())),
                preferred_element_type=jnp.float32) + bias
            m = jnp.max(st, axis=1, keepdims=True)
            p = jnp.exp2((st - m) * EXP2_SCALE)
            z = jnp.sum(p, axis=1, keepdims=True)
            fu = jnp.dot(p.astype(jnp.bfloat16), kh,
                         preferred_element_type=jnp.float32)
            f_ref[pl.ds(r0, BLK), hd * HD:(hd + 1) * HD] = (
                fu / z).astype(jnp.bfloat16)
        return carry

    jax.lax.fori_loop(0, NBLK, body, 0, unroll=4)

    out_ref[...] = (jnp.dot(f_ref[...], wout_ref[...],
                            preferred_element_type=jnp.float32)
                    + bout_ref[...] + x_ref[...])


@jax.jit
def kernel(x, W_in, W_out, b_out):
    x2 = x.reshape(S, D)
    out = pl.pallas_call(
        _fused_kernel,
        out_shape=jax.ShapeDtypeStruct((S, D), jnp.float32),
        scratch_shapes=[
            pltpu.VMEM((S, D), jnp.bfloat16),
            pltpu.VMEM((S, D), jnp.bfloat16),
            pltpu.VMEM((3, BLK, WIN), jnp.float32),
        ],
    )(x2, W_in.astype(jnp.bfloat16), W_out.astype(jnp.bfloat16),
      b_out.reshape(1, D))
    return out.reshape(1, S, D)


# speedup vs baseline: 1.6011x; 1.2354x over previous
import jax
import jax.numpy as jnp
from jax.experimental import pallas as pl
from jax.experimental.pallas import tpu as pltpu

S = 2048
D = 768
H = 12
HD = 64
K = 64
BLK = 512
WIN = 576
NBLK = S // BLK
SQRT_HD = 8.0
EXP2_SCALE = 1.4426950408889634 / SQRT_HD


def _bias_t(r0, c0):
    rows = r0 + jax.lax.broadcasted_iota(jnp.int32, (BLK, WIN), 0)
    cols = c0 + jax.lax.broadcasted_iota(jnp.int32, (BLK, WIN), 1)
    off = cols - rows
    valid = (off >= -(K // 2)) & (off <= K // 2 - 1)
    rowsf = rows.astype(jnp.float32)
    cnt = jnp.where(cols == 0, jnp.maximum(33.0 - rowsf, 1.0), 1.0)
    cnt = jnp.where(cols == S - 1, jnp.maximum(rowsf - (S - 33.0), 1.0), cnt)
    return jnp.where(valid, SQRT_HD * jnp.log(cnt), -1e30)


def _fused_kernel(x_ref, win_ref, wout_ref, bout_ref, out_ref,
                  h_ref, f_ref, bias_ref):
    h_ref[...] = jnp.dot(x_ref[...].astype(jnp.bfloat16), win_ref[...],
                         preferred_element_type=jnp.float32
                         ).astype(jnp.bfloat16)

    bias_ref[0] = _bias_t(0, 0)
    bias_ref[1] = _bias_t(BLK, BLK - K // 2)
    bias_ref[2] = _bias_t(S - BLK, S - WIN)

    def body(i, carry):
        r0 = i * BLK
        c0 = pl.multiple_of(jnp.clip(r0 - K // 2, 0, S - WIN), 32)
        widx = jnp.where(i == 0, 0, jnp.where(i == NBLK - 1, 2, 1))
        q = h_ref[pl.ds(r0, BLK), :]
        kw = h_ref[pl.ds(c0, WIN), :]
        bias = bias_ref[widx]

        for hd in range(H):
            qh = q[:, hd * HD:(hd + 1) * HD]
            kh = kw[:, hd * HD:(hd + 1) * HD]
            st = jax.lax.dot_general(
                qh, kh, (((1,), (1,)), ((), ())),
                preferred_element_type=jnp.float32) + bias
            m = jnp.max(st, axis=1, keepdims=True)
            p = jnp.exp2((st - m) * EXP2_SCALE)
            z = jnp.sum(p, axis=1, keepdims=True)
            fu = jnp.dot(p.astype(jnp.bfloat16), kh,
                         preferred_element_type=jnp.float32)
            f_ref[pl.ds(r0, BLK), hd * HD:(hd + 1) * HD] = (
                fu / z).astype(jnp.bfloat16)
        return carry

    jax.lax.fori_loop(0, NBLK, body, 0, unroll=2)

    out_ref[...] = (jnp.dot(f_ref[...], wout_ref[...],
                            preferred_element_type=jnp.float32)
                    + bout_ref[...] + x_ref[...])


@jax.jit
def kernel(x, W_in, W_out, b_out):
    x2 = x.reshape(S, D)
    out = pl.pallas_call(
        _fused_kernel,
        out_shape=jax.ShapeDtypeStruct((S, D), jnp.float32),
        scratch_shapes=[
            pltpu.VMEM((S, D), jnp.bfloat16),
            pltpu.VMEM((S, D), jnp.bfloat16),
            pltpu.VMEM((3, BLK, WIN), jnp.float32),
        ],
    )(x2, W_in.astype(jnp.bfloat16), W_out.astype(jnp.bfloat16),
      b_out.reshape(1, D))
    return out.reshape(1, S, D)
